# Initial kernel scaffold; baseline (speedup 1.0000x reference)
import jax
import jax.numpy as jnp
from jax.experimental import pallas as pl


def kernel(data, L_real, L_imag, weight, bias):
    raise NotImplementedError("write your pallas kernel here")


# speedup vs baseline: 1.6690x; 1.6690x over previous
import functools

import jax
import jax.numpy as jnp
from jax.experimental import pallas as pl
from jax.experimental.pallas import tpu as pltpu

N = 4096
C = 64
KP1 = 3
BR = 512
BK = 1024


def _body(x_ref, lr_ref, li_ref, w_ref, b_ref, out_ref):
    k = pl.program_id(1)
    i = pl.program_id(2)
    nk = pl.num_programs(1)
    ni = pl.num_programs(2)

    @pl.when((k == 0) & (i == 0))
    def _init():
        out_ref[...] = jnp.zeros_like(out_ref)

    xr = x_ref[0]
    xi = x_ref[1]
    w = w_ref[0]
    p = jnp.dot(xr, w, preferred_element_type=jnp.float32)
    q = jnp.dot(xi, w, preferred_element_type=jnp.float32)
    rt = jnp.concatenate([p, q], axis=1)
    rb = jnp.concatenate([-q, p], axis=1)
    a = lr_ref[0]
    b = li_ref[0]
    acc = jnp.dot(a, rt, preferred_element_type=jnp.float32)
    acc += jnp.dot(b, rb, preferred_element_type=jnp.float32)
    out_ref[...] += acc

    @pl.when((k == nk - 1) & (i == ni - 1))
    def _bias():
        bb = jnp.concatenate([b_ref[...], b_ref[...]], axis=1)
        out_ref[...] += bb


@functools.partial(jax.jit, static_argnames=("interpret",))
def _cheb_conv(data, L_real, L_imag, weight, bias, interpret=False):
    grid = (N // BR, N // BK, KP1)
    out = pl.pallas_call(
        _body,
        grid=grid,
        in_specs=[
            pl.BlockSpec((2, BK, C), lambda r, k, i: (0, k, 0)),
            pl.BlockSpec((1, BR, BK), lambda r, k, i: (i, r, k)),
            pl.BlockSpec((1, BR, BK), lambda r, k, i: (i, r, k)),
            pl.BlockSpec((1, C, C), lambda r, k, i: (i, 0, 0)),
            pl.BlockSpec((1, C), lambda r, k, i: (0, 0)),
        ],
        out_specs=pl.BlockSpec((BR, 2 * C), lambda r, k, i: (r, 0)),
        out_shape=jax.ShapeDtypeStruct((N, 2 * C), jnp.float32),
        compiler_params=pltpu.CompilerParams(
            dimension_semantics=("parallel", "arbitrary", "arbitrary"),
        ),
        interpret=interpret,
    )(data, L_real, L_imag, weight, bias)
    return out[:, :C], out[:, C:]


def kernel(data, L_real, L_imag, weight, bias):
    return _cheb_conv(data, L_real, L_imag, weight, bias)


# speedup vs baseline: 1.9093x; 1.1439x over previous
import functools

import jax
import jax.numpy as jnp
from jax.experimental import pallas as pl
from jax.experimental.pallas import tpu as pltpu

N = 4096
C = 64
KP1 = 3
BR = 512
BK = 1024


def _body(x_ref, lr_ref, li_ref, w_ref, b_ref, out_ref):
    k = pl.program_id(1)
    nk = pl.num_programs(1)

    xr = x_ref[0]
    xi = x_ref[1]
    acc = None
    for i in range(KP1):
        w = w_ref[i]
        p = jnp.dot(xr, w, preferred_element_type=jnp.float32)
        q = jnp.dot(xi, w, preferred_element_type=jnp.float32)
        rt = jnp.concatenate([p, q], axis=1)
        rb = jnp.concatenate([-q, p], axis=1)
        a = lr_ref[i]
        b = li_ref[i]
        part = jnp.dot(a, rt, preferred_element_type=jnp.float32)
        part += jnp.dot(b, rb, preferred_element_type=jnp.float32)
        acc = part if acc is None else acc + part

    @pl.when(k == 0)
    def _first():
        out_ref[...] = acc

    @pl.when(k > 0)
    def _accum():
        out_ref[...] += acc

    @pl.when(k == nk - 1)
    def _bias():
        bb = jnp.concatenate([b_ref[...], b_ref[...]], axis=1)
        out_ref[...] += bb


@functools.partial(jax.jit, static_argnames=("interpret",))
def _cheb_conv(data, L_real, L_imag, weight, bias, interpret=False):
    grid = (N // BR, N // BK)
    out = pl.pallas_call(
        _body,
        grid=grid,
        in_specs=[
            pl.BlockSpec((2, BK, C), lambda r, k: (0, k, 0)),
            pl.BlockSpec((KP1, BR, BK), lambda r, k: (0, r, k)),
            pl.BlockSpec((KP1, BR, BK), lambda r, k: (0, r, k)),
            pl.BlockSpec((KP1, C, C), lambda r, k: (0, 0, 0)),
            pl.BlockSpec((1, C), lambda r, k: (0, 0)),
        ],
        out_specs=pl.BlockSpec((BR, 2 * C), lambda r, k: (r, 0)),
        out_shape=jax.ShapeDtypeStruct((N, 2 * C), jnp.float32),
        compiler_params=pltpu.CompilerParams(
            dimension_semantics=("parallel", "arbitrary"),
        ),
        interpret=interpret,
    )(data, L_real, L_imag, weight, bias)
    return out[:, :C], out[:, C:]


def kernel(data, L_real, L_imag, weight, bias):
    return _cheb_conv(data, L_real, L_imag, weight, bias)


# speedup vs baseline: 2.1936x; 1.1489x over previous
import functools

import jax
import jax.numpy as jnp
from jax.experimental import pallas as pl
from jax.experimental.pallas import tpu as pltpu

N = 4096
C = 64
KP1 = 3
BR = 256
BK = 4096


def _body(x_ref, lr_ref, li_ref, w_ref, b_ref, out_ref):
    k = pl.program_id(1)
    nk = pl.num_programs(1)

    xr = x_ref[0]
    xi = x_ref[1]
    acc = None
    for i in range(KP1):
        w = w_ref[i]
        p = jnp.dot(xr, w, preferred_element_type=jnp.float32)
        q = jnp.dot(xi, w, preferred_element_type=jnp.float32)
        rt = jnp.concatenate([p, q], axis=1)
        rb = jnp.concatenate([-q, p], axis=1)
        a = lr_ref[i]
        b = li_ref[i]
        part = jnp.dot(a, rt, preferred_element_type=jnp.float32)
        part += jnp.dot(b, rb, preferred_element_type=jnp.float32)
        acc = part if acc is None else acc + part

    @pl.when(k == 0)
    def _first():
        out_ref[...] = acc

    @pl.when(k > 0)
    def _accum():
        out_ref[...] += acc

    @pl.when(k == nk - 1)
    def _bias():
        bb = jnp.concatenate([b_ref[...], b_ref[...]], axis=1)
        out_ref[...] += bb


@functools.partial(jax.jit, static_argnames=("interpret",))
def _cheb_conv(data, L_real, L_imag, weight, bias, interpret=False):
    grid = (N // BR, N // BK)
    out = pl.pallas_call(
        _body,
        grid=grid,
        in_specs=[
            pl.BlockSpec((2, BK, C), lambda r, k: (0, k, 0)),
            pl.BlockSpec((KP1, BR, BK), lambda r, k: (0, r, k)),
            pl.BlockSpec((KP1, BR, BK), lambda r, k: (0, r, k)),
            pl.BlockSpec((KP1, C, C), lambda r, k: (0, 0, 0)),
            pl.BlockSpec((1, C), lambda r, k: (0, 0)),
        ],
        out_specs=pl.BlockSpec((BR, 2 * C), lambda r, k: (r, 0)),
        out_shape=jax.ShapeDtypeStruct((N, 2 * C), jnp.float32),
        compiler_params=pltpu.CompilerParams(
            dimension_semantics=("parallel", "arbitrary"),
        ),
        interpret=interpret,
    )(data, L_real, L_imag, weight, bias)
    return out[:, :C], out[:, C:]


def kernel(data, L_real, L_imag, weight, bias):
    return _cheb_conv(data, L_real, L_imag, weight, bias)
